# baseline (speedup 1.0000x reference)
import functools

import jax
import jax.numpy as jnp
from jax import lax
from jax.experimental import pallas as pl
from jax.experimental.pallas import tpu as pltpu
from jax.experimental.pallas import tpu_sc as plsc

B = 4096
T = 200
D = 64
NUM_CORES = 2
NUM_SUBCORES = 16
NW = NUM_CORES * NUM_SUBCORES
ROWS_PER_W = B // NW
TA = 104
TB = T - TA


def _emb_body(ids_hbm, tok_hbm, pos_hbm, out_hbm,
              pos_a, pos_b, idx_a, idx_b, rows_a, rows_b, sem):
    wid = lax.axis_index("s") * NUM_CORES + lax.axis_index("c")

    pltpu.sync_copy(pos_hbm.at[pl.ds(0, TA)], pos_a)
    pltpu.sync_copy(pos_hbm.at[pl.ds(TA, TB)], pos_b)

    def row_body(j, carry):
        base = (wid * ROWS_PER_W + j) * T
        pltpu.sync_copy(ids_hbm.at[pl.ds(base, TA)], idx_a)
        pltpu.sync_copy(ids_hbm.at[pl.ds(base + TA, TB)], idx_b)
        cp_a = pltpu.async_copy(tok_hbm.at[idx_a], rows_a, sem)
        cp_b = pltpu.async_copy(tok_hbm.at[idx_b], rows_b, sem)
        cp_a.wait()
        cp_b.wait()

        def add_a(r, c2):
            for c in range(D // 16):
                s = pl.ds(c * 16, 16)
                rows_a[r, s] = rows_a[r, s] + pos_a[r, s]
            return c2

        def add_b(r, c2):
            for c in range(D // 16):
                s = pl.ds(c * 16, 16)
                rows_b[r, s] = rows_b[r, s] + pos_b[r, s]
            return c2

        lax.fori_loop(0, TA, add_a, 0, unroll=2)
        lax.fori_loop(0, TB, add_b, 0, unroll=2)

        pltpu.sync_copy(rows_a, out_hbm.at[pl.ds(base, TA)])
        pltpu.sync_copy(rows_b, out_hbm.at[pl.ds(base + TA, TB)])
        return carry

    lax.fori_loop(0, ROWS_PER_W, row_body, 0)


@jax.jit
def _emb(ids_flat, token_emb, pos_emb):
    mesh = plsc.VectorSubcoreMesh(core_axis_name="c", subcore_axis_name="s")
    kern = functools.partial(
        pl.kernel,
        out_type=jax.ShapeDtypeStruct((B * T, D), jnp.float32),
        mesh=mesh,
        scratch_types=[
            pltpu.VMEM((TA, D), jnp.float32),
            pltpu.VMEM((TB, D), jnp.float32),
            pltpu.VMEM((TA,), jnp.int32),
            pltpu.VMEM((TB,), jnp.int32),
            pltpu.VMEM((TA, D), jnp.float32),
            pltpu.VMEM((TB, D), jnp.float32),
            pltpu.SemaphoreType.DMA,
        ],
        compiler_params=pltpu.CompilerParams(use_tc_tiling_on_sc=False),
    )(_emb_body)
    return kern(ids_flat, token_emb, pos_emb)


def kernel(input_ids, token_emb, pos_emb):
    ids_flat = input_ids.astype(jnp.int32).reshape(B * T)
    out = _emb(ids_flat, token_emb, pos_emb)
    return out.reshape(B, T, D)


# speedup vs baseline: 1.5196x; 1.5196x over previous
import functools

import jax
import jax.numpy as jnp
from jax import lax
from jax.experimental import pallas as pl
from jax.experimental.pallas import tpu as pltpu
from jax.experimental.pallas import tpu_sc as plsc

B = 4096
T = 200
D = 64
NUM_CORES = 2
NUM_SUBCORES = 16
NW = NUM_CORES * NUM_SUBCORES
ROWS_PER_W = B // NW
TA = 104
TB = T - TA
NBUF = 4


def _emb_body(ids_hbm, tok_hbm, pos_hbm, out_hbm,
              idx_all, pos_a, pos_b,
              ra0, rb0, ra1, rb1, ra2, rb2, ra3, rb3,
              gs0, gs1, gs2, gs3, os0, os1, os2, os3):
    wid = lax.axis_index("s") * NUM_CORES + lax.axis_index("c")
    wbase = wid * (ROWS_PER_W * T)

    ras = (ra0, ra1, ra2, ra3)
    rbs = (rb0, rb1, rb2, rb3)
    gss = (gs0, gs1, gs2, gs3)
    oss = (os0, os1, os2, os3)

    pltpu.sync_copy(ids_hbm.at[pl.ds(wbase, ROWS_PER_W * T)], idx_all)
    pltpu.sync_copy(pos_hbm.at[pl.ds(0, TA)], pos_a)
    pltpu.sync_copy(pos_hbm.at[pl.ds(TA, TB)], pos_b)

    def start_gather(p, j):
        off = j * T
        pltpu.async_copy(tok_hbm.at[idx_all.at[pl.ds(off, TA)]], ras[p], gss[p])
        pltpu.async_copy(tok_hbm.at[idx_all.at[pl.ds(off + TA, TB)]], rbs[p], gss[p])

    def wait_gather(p):
        pltpu.make_async_copy(tok_hbm.at[idx_all.at[pl.ds(0, TA)]], ras[p], gss[p]).wait()
        pltpu.make_async_copy(tok_hbm.at[idx_all.at[pl.ds(0, TB)]], rbs[p], gss[p]).wait()

    def wait_out(p):
        pltpu.make_async_copy(ras[p], out_hbm.at[pl.ds(0, TA)], oss[p]).wait()
        pltpu.make_async_copy(rbs[p], out_hbm.at[pl.ds(0, TB)], oss[p]).wait()

    def finish_row(p, j):
        wait_gather(p)

        def add_a(r, c):
            for cc in range(D // 16):
                s = pl.ds(cc * 16, 16)
                plsc.addupdate(ras[p].at[r, s], pos_a[r, s])
            return c

        def add_b(r, c):
            for cc in range(D // 16):
                s = pl.ds(cc * 16, 16)
                plsc.addupdate(rbs[p].at[r, s], pos_b[r, s])
            return c

        lax.fori_loop(0, TA, add_a, 0, unroll=4)
        lax.fori_loop(0, TB, add_b, 0, unroll=4)
        off = wbase + j * T
        pltpu.async_copy(ras[p], out_hbm.at[pl.ds(off, TA)], oss[p])
        pltpu.async_copy(rbs[p], out_hbm.at[pl.ds(off + TA, TB)], oss[p])

    start_gather(0, 0)
    start_gather(1, 1)
    finish_row(0, 0)
    start_gather(2, 2)
    finish_row(1, 1)
    start_gather(3, 3)
    finish_row(2, 2)

    def body4(i, carry):
        for k in range(NBUF):
            j = 4 * i + 3 + k
            pnext = k
            p = (k + 3) % 4

            @pl.when(j + 1 < ROWS_PER_W)
            def _():
                wait_out(pnext)
                start_gather(pnext, j + 1)

            finish_row(p, j)
        return carry

    lax.fori_loop(0, (ROWS_PER_W - 3) // 4, body4, 0)
    j_last = ROWS_PER_W - 1
    finish_row((j_last % NBUF), j_last)

    for p in range(NBUF):
        wait_out(p)


@jax.jit
def _emb(ids_flat, token_emb, pos_emb):
    mesh = plsc.VectorSubcoreMesh(core_axis_name="c", subcore_axis_name="s")
    kern = functools.partial(
        pl.kernel,
        out_type=jax.ShapeDtypeStruct((B * T, D), jnp.float32),
        mesh=mesh,
        scratch_types=[
            pltpu.VMEM((ROWS_PER_W * T,), jnp.int32),
            pltpu.VMEM((TA, D), jnp.float32),
            pltpu.VMEM((TB, D), jnp.float32),
        ] + [
            buf
            for _ in range(NBUF)
            for buf in (pltpu.VMEM((TA, D), jnp.float32),
                        pltpu.VMEM((TB, D), jnp.float32))
        ] + [pltpu.SemaphoreType.DMA] * (2 * NBUF),
        compiler_params=pltpu.CompilerParams(use_tc_tiling_on_sc=False),
    )(_emb_body)
    return kern(ids_flat, token_emb, pos_emb)


def kernel(input_ids, token_emb, pos_emb):
    ids_flat = input_ids.astype(jnp.int32).reshape(B * T)
    out = _emb(ids_flat, token_emb, pos_emb)
    return out.reshape(B, T, D)


# speedup vs baseline: 1.5268x; 1.0048x over previous
import functools

import jax
import jax.numpy as jnp
from jax import lax
from jax.experimental import pallas as pl
from jax.experimental.pallas import tpu as pltpu
from jax.experimental.pallas import tpu_sc as plsc

B = 4096
T = 200
D = 64
NUM_CORES = 2
NUM_SUBCORES = 16
NW = NUM_CORES * NUM_SUBCORES
ROWS_PER_W = B // NW
TA = 104
TB = T - TA
NBUF = 4


def _emb_body(ids_hbm, tok_hbm, pos_hbm, out_hbm,
              idx_all, pos_sh,
              ra0, rb0, ra1, rb1, ra2, rb2, ra3, rb3,
              gs0, gs1, gs2, gs3, os0, os1, os2, os3):
    sid = lax.axis_index("s")
    wid = sid * NUM_CORES + lax.axis_index("c")
    wbase = wid * (ROWS_PER_W * T)

    ras = (ra0, ra1, ra2, ra3)
    rbs = (rb0, rb1, rb2, rb3)
    gss = (gs0, gs1, gs2, gs3)
    oss = (os0, os1, os2, os3)

    pltpu.sync_copy(ids_hbm.at[pl.ds(wbase, ROWS_PER_W * T)], idx_all)

    @pl.when(sid == 0)
    def _():
        pltpu.sync_copy(pos_hbm, pos_sh)

    plsc.subcore_barrier()

    def start_gather(p, j):
        off = j * T
        pltpu.sync_copy(pos_sh.at[pl.ds(0, TA)], ras[p])
        pltpu.sync_copy(pos_sh.at[pl.ds(TA, TB)], rbs[p])
        pltpu.async_copy(tok_hbm.at[idx_all.at[pl.ds(off, TA)]], ras[p], gss[p],
                         add=True)
        pltpu.async_copy(tok_hbm.at[idx_all.at[pl.ds(off + TA, TB)]], rbs[p], gss[p],
                         add=True)

    def wait_gather(p):
        pltpu.make_async_copy(tok_hbm.at[idx_all.at[pl.ds(0, TA)]], ras[p], gss[p]).wait()
        pltpu.make_async_copy(tok_hbm.at[idx_all.at[pl.ds(0, TB)]], rbs[p], gss[p]).wait()

    def wait_out(p):
        pltpu.make_async_copy(ras[p], out_hbm.at[pl.ds(0, TA)], oss[p]).wait()
        pltpu.make_async_copy(rbs[p], out_hbm.at[pl.ds(0, TB)], oss[p]).wait()

    def finish_row(p, j):
        wait_gather(p)
        off = wbase + j * T
        pltpu.async_copy(ras[p], out_hbm.at[pl.ds(off, TA)], oss[p])
        pltpu.async_copy(rbs[p], out_hbm.at[pl.ds(off + TA, TB)], oss[p])

    start_gather(0, 0)
    start_gather(1, 1)
    finish_row(0, 0)
    start_gather(2, 2)
    finish_row(1, 1)
    start_gather(3, 3)
    finish_row(2, 2)

    def body4(i, carry):
        for k in range(NBUF):
            j = 4 * i + 3 + k
            pnext = k
            p = (k + 3) % 4

            @pl.when(j + 1 < ROWS_PER_W)
            def _():
                wait_out(pnext)
                start_gather(pnext, j + 1)

            finish_row(p, j)
        return carry

    lax.fori_loop(0, (ROWS_PER_W - 3) // 4, body4, 0)
    j_last = ROWS_PER_W - 1
    finish_row((j_last % NBUF), j_last)

    for p in range(NBUF):
        wait_out(p)


@jax.jit
def _emb(ids_flat, token_emb, pos_emb):
    mesh = plsc.VectorSubcoreMesh(core_axis_name="c", subcore_axis_name="s")
    kern = functools.partial(
        pl.kernel,
        out_type=jax.ShapeDtypeStruct((B * T, D), jnp.float32),
        mesh=mesh,
        scratch_types=[
            pltpu.VMEM((ROWS_PER_W * T,), jnp.int32),
            pltpu.VMEM_SHARED((T, D), jnp.float32),
        ] + [
            buf
            for _ in range(NBUF)
            for buf in (pltpu.VMEM((TA, D), jnp.float32),
                        pltpu.VMEM((TB, D), jnp.float32))
        ] + [pltpu.SemaphoreType.DMA] * (2 * NBUF),
        compiler_params=pltpu.CompilerParams(use_tc_tiling_on_sc=False),
    )(_emb_body)
    return kern(ids_flat, token_emb, pos_emb)


def kernel(input_ids, token_emb, pos_emb):
    ids_flat = input_ids.astype(jnp.int32).reshape(B * T)
    out = _emb(ids_flat, token_emb, pos_emb)
    return out.reshape(B, T, D)
